# baseline (speedup 1.0000x reference)
import functools

import jax
import jax.numpy as jnp
from jax import lax
from jax.experimental import pallas as pl
from jax.experimental.pallas import tpu as pltpu
from jax.experimental.pallas import tpu_sc as plsc

D = 200
B = 16384

NC = 2
NS = 16
L = 16
NW = NC * NS
PER_W = B // NW
CB = 16
N_CHUNK = PER_W // CB

FULL_CHUNKS = D // L
TAIL_OFF = D - L
ECOLS = 4 * D
RCOLS = 6 * D
TCOLS = 4 * D

_ROW_BUFS = [
    ((CB, ECOLS), jnp.float32),
    ((CB, ECOLS), jnp.float32),
    ((CB, RCOLS), jnp.float32),
    ((CB, TCOLS), jnp.float32),
]


def _score_kernel(ecat, rcat, tcat, s_idx, r_idx, o_idx, t_idx, out,
                  s_iv, r_iv, o_iv, t_iv, bufs_a, bufs_b, out_v,
                  sem_a, sem_b, idx_sem):
    wid = lax.axis_index("s") * NC + lax.axis_index("c")
    base = pl.multiple_of(wid * PER_W, 8)

    lane = lax.broadcasted_iota(jnp.int32, (L,), 0)
    tail_mask = lane >= (L - (D - FULL_CHUNKS * L))
    perms = [((lane + sh) % L)[:, None] for sh in (8, 4, 2, 1)]
    _dn = lax.GatherDimensionNumbers(
        offset_dims=(), collapsed_slice_dims=(0,), start_index_map=(0,))

    def _lane_sum(v):
        for p in perms:
            v = v + lax.gather(v, p, _dn, slice_sizes=(1,),
                               mode=lax.GatherScatterMode.PROMISE_IN_BOUNDS)
        return v

    cps = [pltpu.async_copy(s_idx.at[pl.ds(base, PER_W)], s_iv, idx_sem),
           pltpu.async_copy(r_idx.at[pl.ds(base, PER_W)], r_iv, idx_sem),
           pltpu.async_copy(o_idx.at[pl.ds(base, PER_W)], o_iv, idx_sem),
           pltpu.async_copy(t_idx.at[pl.ds(base, PER_W)], t_iv, idx_sem)]
    for cp in cps:
        cp.wait()

    def descriptors(ch, bufs, sem):
        off = pl.multiple_of(ch * CB, 8)
        b_s, b_o, b_r, b_t = bufs
        return [
            pltpu.make_async_copy(ecat.at[s_iv.at[pl.ds(off, CB)]], b_s, sem),
            pltpu.make_async_copy(ecat.at[o_iv.at[pl.ds(off, CB)]], b_o, sem),
            pltpu.make_async_copy(rcat.at[r_iv.at[pl.ds(off, CB)]], b_r, sem),
            pltpu.make_async_copy(tcat.at[t_iv.at[pl.ds(off, CB)]], b_t, sem),
        ]

    def gathers(ch, bufs, sem):
        for cp in descriptors(ch, bufs, sem):
            cp.start()

    def wait_gathers(ch, bufs, sem):
        for cp in descriptors(ch, bufs, sem):
            cp.wait()

    def compute(ch, bufs):
        b_s, b_o, b_r, b_t = bufs

        def term_sums(e, d0, tail):
            s_re = b_s[e, pl.ds(d0, L)]
            s_im = b_s[e, pl.ds(D + d0, L)]
            s2_re = b_s[e, pl.ds(2 * D + d0, L)]
            s2_im = b_s[e, pl.ds(3 * D + d0, L)]
            o_re = b_o[e, pl.ds(d0, L)]
            o_im = b_o[e, pl.ds(D + d0, L)]
            o2_re = b_o[e, pl.ds(2 * D + d0, L)]
            o2_im = b_o[e, pl.ds(3 * D + d0, L)]
            r_re = b_r[e, pl.ds(d0, L)]
            r_im = b_r[e, pl.ds(D + d0, L)]
            rs_re = b_r[e, pl.ds(2 * D + d0, L)]
            rs_im = b_r[e, pl.ds(3 * D + d0, L)]
            ro_re = b_r[e, pl.ds(4 * D + d0, L)]
            ro_im = b_r[e, pl.ds(5 * D + d0, L)]
            ts_re = b_t[e, pl.ds(d0, L)]
            ts_im = b_t[e, pl.ds(D + d0, L)]
            to_re = b_t[e, pl.ds(2 * D + d0, L)]
            to_im = b_t[e, pl.ds(3 * D + d0, L)]
            sro = ((s_im * r_re + s_re * r_im) * o_im
                   + (s_re * r_re - s_im * r_im) * o_re)
            srt = ((s_im * rs_re + s_re * rs_im) * ts_im
                   + (s_re * rs_re - s_im * rs_im) * ts_re)
            ort = ((o_im * ro_re + o_re * ro_im) * to_im
                   + (o_re * ro_re - o_im * ro_im) * to_re)
            sot = ((s2_im * ts_re + s2_re * ts_im) * o2_im
                   + (s2_re * ts_re - s2_im * ts_im) * o2_re)
            w5 = srt + ort + sot
            if tail:
                sro = jnp.where(tail_mask, sro, 0.0)
                w5 = jnp.where(tail_mask, w5, 0.0)
            return sro, w5

        def elem_body(e, vec):
            def dchunk(c, accs):
                a1, a5 = accs
                sro, w5 = term_sums(e, c * L, False)
                return (a1 + sro, a5 + w5)

            acc1, acc5 = lax.fori_loop(
                0, FULL_CHUNKS, dchunk,
                (jnp.zeros((L,), jnp.float32), jnp.zeros((L,), jnp.float32)),
                unroll=False)
            sro, w5 = term_sums(e, TAIL_OFF, True)
            tot = _lane_sum((acc1 + sro) + 5.0 * (acc5 + w5))
            return jnp.where(lane == e, tot, vec)

        vec = lax.fori_loop(0, CB, elem_body,
                            jnp.zeros((L,), jnp.float32), unroll=False)
        out_v[pl.ds(pl.multiple_of(ch * CB, L), L)] = vec

    gathers(0, bufs_a, sem_a)
    gathers(1, bufs_b, sem_b)

    def pair_body(p, _):
        ch = 2 * p
        wait_gathers(ch, bufs_a, sem_a)
        compute(ch, bufs_a)

        @pl.when(ch + 2 < N_CHUNK)
        def _():
            gathers(ch + 2, bufs_a, sem_a)

        wait_gathers(ch + 1, bufs_b, sem_b)
        compute(ch + 1, bufs_b)

        @pl.when(ch + 3 < N_CHUNK)
        def _():
            gathers(ch + 3, bufs_b, sem_b)

        return ()

    lax.fori_loop(0, N_CHUNK // 2, pair_body, (), unroll=False)
    pltpu.sync_copy(out_v, out.at[pl.ds(base, PER_W)])


@jax.jit
def _timeplex_sc(ecat, rcat, tcat, s, r, o, t):
    mesh = plsc.VectorSubcoreMesh(core_axis_name="c", subcore_axis_name="s")
    kfn = functools.partial(
        pl.kernel,
        mesh=mesh,
        out_type=jax.ShapeDtypeStruct((B,), jnp.float32),
        scratch_types=[
            pltpu.VMEM((PER_W,), jnp.int32),
            pltpu.VMEM((PER_W,), jnp.int32),
            pltpu.VMEM((PER_W,), jnp.int32),
            pltpu.VMEM((PER_W,), jnp.int32),
            [pltpu.VMEM(shape, dt) for shape, dt in _ROW_BUFS],
            [pltpu.VMEM(shape, dt) for shape, dt in _ROW_BUFS],
            pltpu.VMEM((PER_W,), jnp.float32),
            pltpu.SemaphoreType.DMA,
            pltpu.SemaphoreType.DMA,
            pltpu.SemaphoreType.DMA,
        ],
        compiler_params=pltpu.CompilerParams(use_tc_tiling_on_sc=False),
    )(_score_kernel)
    return kfn(ecat, rcat, tcat, s, r, o, t)


def kernel(E_re, E_im, E2_re, E2_im, R_re, R_im, Rs_re, Rs_im, Ro_re,
           Ro_im, Ts_re, Ts_im, To_re, To_im, s, r, o, t):
    ecat = jnp.concatenate([E_re, E_im, E2_re, E2_im], axis=1)
    rcat = jnp.concatenate([R_re, R_im, Rs_re, Rs_im, Ro_re, Ro_im], axis=1)
    tcat = jnp.concatenate([Ts_re, Ts_im, To_re, To_im], axis=1)
    return _timeplex_sc(ecat, rcat, tcat, s, r, o, t)


# speedup vs baseline: 2.5119x; 2.5119x over previous
import functools

import jax
import jax.numpy as jnp
from jax import lax
from jax.experimental import pallas as pl
from jax.experimental.pallas import tpu as pltpu
from jax.experimental.pallas import tpu_sc as plsc

D = 200
B = 16384

NC = 2
NS = 16
L = 16
NW = NC * NS
PER_W = B // NW
CB = 8
N_CHUNK = PER_W // CB

HEAD = 128
TAIL = D - HEAD
HEAD_CHUNKS = HEAD // L
TAIL_CHUNKS = TAIL // L
TAIL_MOFF = TAIL - L

_ROW_BUFS = (
    [((CB, HEAD), jnp.float32)] * 8
    + [((CB, 512), jnp.float32)] * 2
    + [((CB, 1536), jnp.float32), ((CB, 1024), jnp.float32)]
)


def _score_kernel(e_re, e_im, e2_re, e2_im, etailp, rcatp, tcatp,
                  s_idx, r_idx, o_idx, t_idx, out,
                  s_iv, r_iv, o_iv, t_iv, bufs_a, bufs_b, out_v,
                  sem_a, sem_b, idx_sem):
    wid = lax.axis_index("s") * NC + lax.axis_index("c")
    base = pl.multiple_of(wid * PER_W, 8)

    lane = lax.broadcasted_iota(jnp.int32, (L,), 0)
    tail_mask = lane >= (L - (TAIL - TAIL_CHUNKS * L))
    perms = [((lane + sh) % L)[:, None] for sh in (8, 4, 2, 1)]
    _dn = lax.GatherDimensionNumbers(
        offset_dims=(), collapsed_slice_dims=(0,), start_index_map=(0,))

    def _lane_sum(v):
        for p in perms:
            v = v + lax.gather(v, p, _dn, slice_sizes=(1,),
                               mode=lax.GatherScatterMode.PROMISE_IN_BOUNDS)
        return v

    cps = [pltpu.async_copy(s_idx.at[pl.ds(base, PER_W)], s_iv, idx_sem),
           pltpu.async_copy(r_idx.at[pl.ds(base, PER_W)], r_iv, idx_sem),
           pltpu.async_copy(o_idx.at[pl.ds(base, PER_W)], o_iv, idx_sem),
           pltpu.async_copy(t_idx.at[pl.ds(base, PER_W)], t_iv, idx_sem)]
    for cp in cps:
        cp.wait()

    def descriptors(ch, bufs, sem):
        off = pl.multiple_of(ch * CB, 8)
        (b_sre, b_sim, b_s2re, b_s2im, b_ore, b_oim, b_o2re, b_o2im,
         b_st, b_ot, b_r, b_t) = bufs
        s_i = s_iv.at[pl.ds(off, CB)]
        o_i = o_iv.at[pl.ds(off, CB)]
        hd = pl.ds(0, HEAD)
        return [
            pltpu.make_async_copy(e_re.at[s_i, hd], b_sre, sem),
            pltpu.make_async_copy(e_im.at[s_i, hd], b_sim, sem),
            pltpu.make_async_copy(e2_re.at[s_i, hd], b_s2re, sem),
            pltpu.make_async_copy(e2_im.at[s_i, hd], b_s2im, sem),
            pltpu.make_async_copy(e_re.at[o_i, hd], b_ore, sem),
            pltpu.make_async_copy(e_im.at[o_i, hd], b_oim, sem),
            pltpu.make_async_copy(e2_re.at[o_i, hd], b_o2re, sem),
            pltpu.make_async_copy(e2_im.at[o_i, hd], b_o2im, sem),
            pltpu.make_async_copy(etailp.at[s_i], b_st, sem),
            pltpu.make_async_copy(etailp.at[o_i], b_ot, sem),
            pltpu.make_async_copy(rcatp.at[r_iv.at[pl.ds(off, CB)]], b_r, sem),
            pltpu.make_async_copy(tcatp.at[t_iv.at[pl.ds(off, CB)]], b_t, sem),
        ]

    def gathers(ch, bufs, sem):
        for cp in descriptors(ch, bufs, sem):
            cp.start()

    def wait_gathers(ch, bufs, sem):
        for cp in descriptors(ch, bufs, sem):
            cp.wait()

    def compute(ch, bufs, lane_base, vec):
        (b_sre, b_sim, b_s2re, b_s2im, b_ore, b_oim, b_o2re, b_o2im,
         b_st, b_ot, b_r, b_t) = bufs

        def cross_terms(e, s_re, s_im, s2_re, s2_im, o_re, o_im, o2_re,
                        o2_im, rt0, tail):
            r_re = b_r[e, pl.ds(rt0, L)]
            r_im = b_r[e, pl.ds(256 + rt0, L)]
            rs_re = b_r[e, pl.ds(512 + rt0, L)]
            rs_im = b_r[e, pl.ds(768 + rt0, L)]
            ro_re = b_r[e, pl.ds(1024 + rt0, L)]
            ro_im = b_r[e, pl.ds(1280 + rt0, L)]
            ts_re = b_t[e, pl.ds(rt0, L)]
            ts_im = b_t[e, pl.ds(256 + rt0, L)]
            to_re = b_t[e, pl.ds(512 + rt0, L)]
            to_im = b_t[e, pl.ds(768 + rt0, L)]
            sro = ((s_im * r_re + s_re * r_im) * o_im
                   + (s_re * r_re - s_im * r_im) * o_re)
            srt = ((s_im * rs_re + s_re * rs_im) * ts_im
                   + (s_re * rs_re - s_im * rs_im) * ts_re)
            ort = ((o_im * ro_re + o_re * ro_im) * to_im
                   + (o_re * ro_re - o_im * ro_im) * to_re)
            sot = ((s2_im * ts_re + s2_re * ts_im) * o2_im
                   + (s2_re * ts_re - s2_im * ts_im) * o2_re)
            w5 = srt + ort + sot
            if tail:
                sro = jnp.where(tail_mask, sro, 0.0)
                w5 = jnp.where(tail_mask, w5, 0.0)
            return sro, w5

        def head_terms(e, c, accs):
            d0 = c * L
            sro, w5 = cross_terms(
                e,
                b_sre[e, pl.ds(d0, L)], b_sim[e, pl.ds(d0, L)],
                b_s2re[e, pl.ds(d0, L)], b_s2im[e, pl.ds(d0, L)],
                b_ore[e, pl.ds(d0, L)], b_oim[e, pl.ds(d0, L)],
                b_o2re[e, pl.ds(d0, L)], b_o2im[e, pl.ds(d0, L)],
                d0, False)
            return (accs[0] + sro, accs[1] + w5)

        def tail_terms(e, toff, tail, accs):
            sro, w5 = cross_terms(
                e,
                b_st[e, pl.ds(toff, L)], b_st[e, pl.ds(128 + toff, L)],
                b_st[e, pl.ds(256 + toff, L)], b_st[e, pl.ds(384 + toff, L)],
                b_ot[e, pl.ds(toff, L)], b_ot[e, pl.ds(128 + toff, L)],
                b_ot[e, pl.ds(256 + toff, L)], b_ot[e, pl.ds(384 + toff, L)],
                HEAD + toff, tail)
            if tail:
                return (accs[0] + sro, accs[1] + w5)
            return (accs[0] + sro, accs[1] + w5)

        def elem_body(e, vec):
            z = (jnp.zeros((L,), jnp.float32), jnp.zeros((L,), jnp.float32))
            accs = lax.fori_loop(
                0, HEAD_CHUNKS, lambda c, a: head_terms(e, c, a), z,
                unroll=False)
            accs = lax.fori_loop(
                0, TAIL_CHUNKS, lambda c, a: tail_terms(e, c * L, False, a),
                accs, unroll=False)
            accs = tail_terms(e, TAIL_MOFF, True, accs)
            tot = _lane_sum(accs[0] + 5.0 * accs[1])
            return jnp.where(lane == e + lane_base, tot, vec)

        return lax.fori_loop(0, CB, elem_body, vec, unroll=False)

    gathers(0, bufs_a, sem_a)
    gathers(1, bufs_b, sem_b)

    def pair_body(p, _):
        ch = 2 * p
        wait_gathers(ch, bufs_a, sem_a)
        vec = compute(ch, bufs_a, 0, jnp.zeros((L,), jnp.float32))

        @pl.when(ch + 2 < N_CHUNK)
        def _():
            gathers(ch + 2, bufs_a, sem_a)

        wait_gathers(ch + 1, bufs_b, sem_b)
        vec = compute(ch + 1, bufs_b, CB, vec)
        out_v[pl.ds(pl.multiple_of(p * L, L), L)] = vec

        @pl.when(ch + 3 < N_CHUNK)
        def _():
            gathers(ch + 3, bufs_b, sem_b)

        return ()

    lax.fori_loop(0, N_CHUNK // 2, pair_body, (), unroll=False)
    pltpu.sync_copy(out_v, out.at[pl.ds(base, PER_W)])


@jax.jit
def _timeplex_sc(e_re, e_im, e2_re, e2_im, etailp, rcatp, tcatp, s, r, o, t):
    mesh = plsc.VectorSubcoreMesh(core_axis_name="c", subcore_axis_name="s")
    kfn = functools.partial(
        pl.kernel,
        mesh=mesh,
        out_type=jax.ShapeDtypeStruct((B,), jnp.float32),
        scratch_types=[
            pltpu.VMEM((PER_W,), jnp.int32),
            pltpu.VMEM((PER_W,), jnp.int32),
            pltpu.VMEM((PER_W,), jnp.int32),
            pltpu.VMEM((PER_W,), jnp.int32),
            [pltpu.VMEM(shape, dt) for shape, dt in _ROW_BUFS],
            [pltpu.VMEM(shape, dt) for shape, dt in _ROW_BUFS],
            pltpu.VMEM((PER_W,), jnp.float32),
            pltpu.SemaphoreType.DMA,
            pltpu.SemaphoreType.DMA,
            pltpu.SemaphoreType.DMA,
        ],
        compiler_params=pltpu.CompilerParams(use_tc_tiling_on_sc=True),
    )(_score_kernel)
    return kfn(e_re, e_im, e2_re, e2_im, etailp, rcatp, tcatp, s, r, o, t)


def kernel(E_re, E_im, E2_re, E2_im, R_re, R_im, Rs_re, Rs_im, Ro_re,
           Ro_im, Ts_re, Ts_im, To_re, To_im, s, r, o, t):
    ze = jnp.zeros((E_re.shape[0], HEAD - TAIL), jnp.float32)
    etailp = jnp.concatenate(
        [E_re[:, HEAD:], ze, E_im[:, HEAD:], ze,
         E2_re[:, HEAD:], ze, E2_im[:, HEAD:], ze], axis=1)
    zr = jnp.zeros((R_re.shape[0], 56), jnp.float32)
    rcatp = jnp.concatenate(
        [R_re, zr, R_im, zr, Rs_re, zr, Rs_im, zr, Ro_re, zr, Ro_im, zr],
        axis=1)
    zt = jnp.zeros((Ts_re.shape[0], 56), jnp.float32)
    tcatp = jnp.concatenate(
        [Ts_re, zt, Ts_im, zt, To_re, zt, To_im, zt], axis=1)
    return _timeplex_sc(E_re, E_im, E2_re, E2_im, etailp, rcatp, tcatp,
                        s, r, o, t)
